# Initial kernel scaffold; baseline (speedup 1.0000x reference)
import jax
import jax.numpy as jnp
from jax.experimental import pallas as pl


def kernel(positions, target_cs, emb_type, emb_res, emb_num, W_edge, W1, W_out, node_types, atom_restypes, atom_numbers, filter_atoms):
    raise NotImplementedError("write your pallas kernel here")


# speedup vs baseline: 40.2050x; 40.2050x over previous
import functools

import jax
import jax.numpy as jnp
from jax.experimental import pallas as pl
from jax.experimental.pallas import tpu as pltpu

_D = 128
_NRBF = 16
_BC = 64
_BN = 512


def _msg_body(pc_ref, pnT_ref, hc_ref, h_ref, We_ref, W1_ref, Wo_ref,
              p_ref, A_ref):
    j = pl.program_id(1)
    nj = pl.num_programs(1)

    @pl.when(j == 0)
    def _init():
        A_ref[...] = jnp.zeros_like(A_ref)

    pc = pc_ref[...]
    pnT = pnT_ref[...]
    pc2 = jnp.sum(pc * pc, axis=1, keepdims=True)
    pn2 = jnp.sum(pnT * pnT, axis=0, keepdims=True)
    cross = jnp.dot(pc.astype(jnp.bfloat16), pnT.astype(jnp.bfloat16),
                    preferred_element_type=jnp.float32)
    d2 = jnp.maximum(pc2 + pn2 - 2.0 * cross, 0.0)
    dm = jnp.sqrt(d2)
    maskf = jnp.where((dm > 0.01) & (dm <= 0.5), 1.0, 0.0)
    pcA = pc * 10.0
    pnA = pnT * 10.0
    r0 = pcA[:, 0:1] - pnA[0:1, :]
    r1 = pcA[:, 1:2] - pnA[1:2, :]
    r2 = pcA[:, 2:3] - pnA[2:3, :]
    dist = jnp.sqrt(r0 * r0 + r1 * r1 + r2 * r2 + 1e-12)
    rows = []
    for k in range(_NRBF):
        mu = 5.0 * k / 15.0
        rows.append(maskf * jnp.exp(-2.0 * (dist - mu) ** 2))
    S = jnp.concatenate(rows, axis=0)
    A_ref[...] += jnp.dot(S, h_ref[...], preferred_element_type=jnp.float32)

    @pl.when(j == nj - 1)
    def _epilogue():
        A = A_ref[...]
        contrib = jnp.zeros((_BC, _D), dtype=jnp.float32)
        for k in range(_NRBF):
            contrib += A[k * _BC:(k + 1) * _BC, :] * We_ref[k:k + 1, :]
        hrow = hc_ref[...] + jax.nn.relu(
            jnp.dot(contrib, W1_ref[...], preferred_element_type=jnp.float32))
        p_ref[...] = jnp.dot(hrow, Wo_ref[...],
                             preferred_element_type=jnp.float32)


def _round_up(x, m):
    return (x + m - 1) // m * m


def kernel(positions, target_cs, emb_type, emb_res, emb_num, W_edge, W1,
           W_out, node_types, atom_restypes, atom_numbers, filter_atoms):
    n = positions.shape[0]
    nf = target_cs.shape[0]
    centers = jnp.where(filter_atoms, size=nf)[0]

    h = (emb_type[node_types] + emb_res[atom_restypes]
         + emb_num[atom_numbers])

    npad = _round_up(n, _BN)
    nfpad = _round_up(nf, _BC)

    pos_pad = jnp.pad(positions, ((0, npad - n), (0, 0)),
                      constant_values=1000.0)
    pnT = pos_pad.T
    h_pad = jnp.pad(h, ((0, npad - n), (0, 0)))
    pc_pad = jnp.pad(positions[centers], ((0, nfpad - nf), (0, 0)),
                     constant_values=1000.0)
    hc_pad = jnp.pad(h[centers], ((0, nfpad - nf), (0, 0)))

    grid = (nfpad // _BC, npad // _BN)
    p = pl.pallas_call(
        _msg_body,
        grid=grid,
        in_specs=[
            pl.BlockSpec((_BC, 3), lambda c, j: (c, 0)),
            pl.BlockSpec((3, _BN), lambda c, j: (0, j)),
            pl.BlockSpec((_BC, _D), lambda c, j: (c, 0)),
            pl.BlockSpec((_BN, _D), lambda c, j: (j, 0)),
            pl.BlockSpec((_NRBF, _D), lambda c, j: (0, 0)),
            pl.BlockSpec((_D, _D), lambda c, j: (0, 0)),
            pl.BlockSpec((_D, 1), lambda c, j: (0, 0)),
        ],
        out_specs=pl.BlockSpec((_BC, 1), lambda c, j: (c, 0)),
        out_shape=jax.ShapeDtypeStruct((nfpad, 1), jnp.float32),
        scratch_shapes=[pltpu.VMEM((_NRBF * _BC, _D), jnp.float32)],
    )(pc_pad, pnT, hc_pad, h_pad, W_edge, W1, W_out)

    p = p[:nf, 0]
    t = target_cs[:, 0]
    nff = jnp.float32(nf)
    energy = (nff * jnp.sum(p * p) - 2.0 * jnp.sum(p) * jnp.sum(t)
              + nff * jnp.sum(t * t))
    return energy
